# Initial kernel scaffold; baseline (speedup 1.0000x reference)
import jax
import jax.numpy as jnp
from jax.experimental import pallas as pl


def kernel(x_feat, x_label, adj1_idx, adj1_val, adj2_idx, adj2_val, Wf, Wl, Gw1, Gb1, Gw2, Gb2, Ow, Ob):
    raise NotImplementedError("write your pallas kernel here")


# speedup vs baseline: 8.2592x; 8.2592x over previous
import functools

import jax
import jax.numpy as jnp
from jax import lax
from jax.experimental import pallas as pl
from jax.experimental.pallas import tpu as pltpu
from jax.experimental.pallas import tpu_sc as plsc

N = 10000
E = 320000
HID = 128
SW = 160
NC = 2
NS = 16
CHUNK = 128
EPW = E // NS
CH = (EPW + CHUNK - 1) // CHUNK
EPWP = CH * CHUNK
NP = 10240
RPW = NP // NS


def _dotT(a, w):
    return lax.dot_general(a, w, (((1,), (1,)), ((), ())),
                           preferred_element_type=jnp.float32)


def _pre_body(xf_ref, xl_ref, Wf_ref, Wl_ref, Gw1_ref, Ow_ref,
              S_ref, D0_ref, O0_ref):
    h0f = _dotT(xf_ref[...], Wf_ref[...])
    h0l = _dotT(xl_ref[...], Wl_ref[...])
    G = Gw1_ref[...]
    O = Ow_ref[...]
    s1 = jnp.concatenate(
        [_dotT(h0f, G[:, 128:256]) + _dotT(h0l, G[:, 512:640]),
         _dotT(h0f, O[:, 128:256]),
         _dotT(h0l, O[:, 128:256])], axis=1)
    s2 = jnp.concatenate(
        [_dotT(h0f, G[:, 256:384]) + _dotT(h0l, G[:, 640:768]),
         _dotT(h0f, O[:, 256:384]),
         _dotT(h0l, O[:, 256:384])], axis=1)
    S_ref[0] = s1
    S_ref[1] = s2
    D0_ref[...] = _dotT(h0f, G[:, 0:128]) + _dotT(h0l, G[:, 384:512])
    O0_ref[...] = jnp.concatenate(
        [_dotT(h0f, O[:, 0:128]), _dotT(h0l, O[:, 0:128])], axis=1)


def _pre_call(xf, xl, Wf, Wl, Gw1, Ow, block=2000):
    grid = (N // block,)
    full = lambda i: (0, 0)
    return pl.pallas_call(
        _pre_body,
        grid=grid,
        in_specs=[
            pl.BlockSpec((block, 128), lambda i: (i, 0)),
            pl.BlockSpec((block, 16), lambda i: (i, 0)),
            pl.BlockSpec((128, 128), full),
            pl.BlockSpec((128, 16), full),
            pl.BlockSpec((128, 768), full),
            pl.BlockSpec((16, 384), full),
        ],
        out_specs=[
            pl.BlockSpec((2, block, SW), lambda i: (0, i, 0)),
            pl.BlockSpec((block, 128), lambda i: (i, 0)),
            pl.BlockSpec((block, 32), lambda i: (i, 0)),
        ],
        out_shape=[
            jax.ShapeDtypeStruct((2, N, SW), jnp.float32),
            jax.ShapeDtypeStruct((N, 128), jnp.float32),
            jax.ShapeDtypeStruct((N, 32), jnp.float32),
        ],
    )(xf, xl, Wf, Wl, Gw1, Ow)


def _sc_body(S_hbm, src_hbm, dst_hbm, val_hbm, T_hbm,
             srcb, dstb, valb, rows, accum, semf, semg):
    c = lax.axis_index("c")
    s = lax.axis_index("s")

    def _fetch(g, b, start):
        op = pltpu.async_copy if start else pltpu.make_async_copy
        return (op(src_hbm.at[c, s, g], srcb.at[b], semf),
                op(dst_hbm.at[c, s, g], dstb.at[b], semf),
                op(val_hbm.at[c, s, g], valb.at[b], semf))

    _fetch(0, 0, True)

    def _zrow(r, _):
        for cc in range(SW // 16):
            rows[r, pl.ds(cc * 16, 16)] = jnp.zeros((16,), jnp.float32)
        return 0
    lax.fori_loop(0, CHUNK, _zrow, 0)
    base = s * RPW
    for k in range(RPW // CHUNK):
        pltpu.sync_copy(rows, accum.at[pl.ds(base + k * CHUNK, CHUNK)])
    plsc.subcore_barrier()

    def _chunk(g, _):
        b = lax.rem(g, 2)
        for d in _fetch(g, b, False):
            d.wait()
        @pl.when(g + 1 < CH)
        def _():
            _fetch(g + 1, 1 - b, True)

        pltpu.async_copy(S_hbm.at[srcb.at[b]], rows, semg).wait()

        def _rowgrp(jj, _):
            vv = valb[b, pl.ds(jj * 16, 16)]
            for k in range(16):
                sv = jnp.full((16,), vv[k], dtype=jnp.float32)
                r = jj * 16 + k
                for cc in range(SW // 16):
                    sl = pl.ds(cc * 16, 16)
                    rows[r, sl] = rows[r, sl] * sv
            return 0
        lax.fori_loop(0, CHUNK // 16, _rowgrp, 0)
        pltpu.sync_copy(rows, accum.at[dstb.at[b]], add=True)
        return 0
    lax.fori_loop(0, CH, _chunk, 0)
    plsc.subcore_barrier()

    pltpu.sync_copy(accum.at[pl.ds(base, RPW)], T_hbm.at[c, pl.ds(base, RPW)])


def _sc_call(S, src, dst, val):
    mesh = plsc.VectorSubcoreMesh(core_axis_name="c", subcore_axis_name="s")
    return pl.kernel(
        _sc_body,
        out_type=jax.ShapeDtypeStruct((2, NP, SW), jnp.float32),
        mesh=mesh,
        scratch_types=[
            pltpu.VMEM((2, CHUNK), jnp.int32),
            pltpu.VMEM((2, CHUNK), jnp.int32),
            pltpu.VMEM((2, CHUNK), jnp.float32),
            pltpu.VMEM((CHUNK, SW), jnp.float32),
            pltpu.VMEM_SHARED((NP, SW), jnp.float32),
            pltpu.SemaphoreType.DMA,
            pltpu.SemaphoreType.DMA,
        ],
        compiler_params=pltpu.CompilerParams(use_tc_tiling_on_sc=False),
    )(S, src, dst, val)


def _post_body(T_ref, D0_ref, O0_ref, Gb1_ref, Gw2_ref, Gb2_ref, Ob_ref,
               out_ref, gate_ref):
    t1 = T_ref[0]
    t2 = T_ref[1]
    relu_in = D0_ref[...] + t1[:, :128] + t2[:, :128] + Gb1_ref[...]
    r = jnp.maximum(relu_in, 0.0)
    z8 = _dotT(r, Gw2_ref[...]) + Gb2_ref[0, 0]
    g8 = jax.nn.sigmoid(z8)
    col = lax.broadcasted_iota(jnp.int32, (384, 8), 1)
    gb = _dotT(g8, (col == 0).astype(jnp.float32))
    g16 = gb[:, 0:16]
    outf = O0_ref[:, 0:16] + t1[:, 128:144] + t2[:, 128:144]
    outl = O0_ref[:, 16:32] + t1[:, 144:160] + t2[:, 144:160]
    out_ref[...] = g16 * outf + (1.0 - g16) * outl + Ob_ref[...]
    gate_ref[...] = gb


def _post_call(T, D0, O0, Gb1, Gw2, Gb2, Ob, block=2000):
    grid = (N // block,)
    full = lambda i: (0, 0)
    return pl.pallas_call(
        _post_body,
        grid=grid,
        in_specs=[
            pl.BlockSpec((2, block, SW), lambda i: (0, i, 0)),
            pl.BlockSpec((block, 128), lambda i: (i, 0)),
            pl.BlockSpec((block, 32), lambda i: (i, 0)),
            pl.BlockSpec((1, 128), full),
            pl.BlockSpec((8, 128), full),
            pl.BlockSpec(memory_space=pltpu.SMEM),
            pl.BlockSpec((1, 16), full),
        ],
        out_specs=[
            pl.BlockSpec((block, 16), lambda i: (i, 0)),
            pl.BlockSpec((block, 384), lambda i: (i, 0)),
        ],
        out_shape=[
            jax.ShapeDtypeStruct((N, 16), jnp.float32),
            jax.ShapeDtypeStruct((N, 384), jnp.float32),
        ],
    )(T, D0, O0, Gb1, Gw2, Gb2, Ob)


_ER = E // CHUNK
_PR = NS * CH

def _prep_body(a1i_ref, a1v_ref, a2i_ref, a2v_ref,
               srcp_ref, dstp_ref, valp_ref):
    zi = jnp.zeros((_PR - _ER, CHUNK), jnp.int32)
    zf = jnp.zeros((_PR - _ER, CHUNK), jnp.float32)
    srcp_ref[0] = jnp.concatenate([a1i_ref[1], zi], axis=0)
    srcp_ref[1] = jnp.concatenate([a2i_ref[1] + N, zi], axis=0)
    dstp_ref[0] = jnp.concatenate([a1i_ref[0], zi], axis=0)
    dstp_ref[1] = jnp.concatenate([a2i_ref[0], zi], axis=0)
    valp_ref[0] = jnp.concatenate([a1v_ref[...], zf], axis=0)
    valp_ref[1] = jnp.concatenate([a2v_ref[...], zf], axis=0)


def _prep_call(a1i, a1v, a2i, a2v):
    full2 = lambda: (0, 0)
    full3 = lambda: (0, 0, 0)
    return pl.pallas_call(
        _prep_body,
        in_specs=[
            pl.BlockSpec((2, _ER, CHUNK), full3),
            pl.BlockSpec((_ER, CHUNK), full2),
            pl.BlockSpec((2, _ER, CHUNK), full3),
            pl.BlockSpec((_ER, CHUNK), full2),
        ],
        out_specs=[
            pl.BlockSpec((2, _PR, CHUNK), full3),
            pl.BlockSpec((2, _PR, CHUNK), full3),
            pl.BlockSpec((2, _PR, CHUNK), full3),
        ],
        out_shape=[
            jax.ShapeDtypeStruct((2, _PR, CHUNK), jnp.int32),
            jax.ShapeDtypeStruct((2, _PR, CHUNK), jnp.int32),
            jax.ShapeDtypeStruct((2, _PR, CHUNK), jnp.float32),
        ],
    )(a1i, a1v, a2i, a2v)


def kernel(x_feat, x_label, adj1_idx, adj1_val, adj2_idx, adj2_val,
           Wf, Wl, Gw1, Gb1, Gw2, Gb2, Ow, Ob):
    S, D0, O0 = _pre_call(x_feat, x_label, Wf, Wl, Gw1, Ow)

    srcp, dstp, valp = _prep_call(
        adj1_idx.reshape(2, _ER, CHUNK), adj1_val.reshape(_ER, CHUNK),
        adj2_idx.reshape(2, _ER, CHUNK), adj2_val.reshape(_ER, CHUNK))
    shp = (2, NS, CH, CHUNK)
    T = _sc_call(S.reshape(2 * N, SW), srcp.reshape(shp),
                 dstp.reshape(shp), valp.reshape(shp))

    Gw2p = jnp.concatenate([Gw2, jnp.zeros((7, 128), jnp.float32)], axis=0)
    out, gate = _post_call(T, D0, O0,
                           Gb1.reshape(1, 128), Gw2p,
                           Gb2.reshape(1, 1), Ob.reshape(1, 16))
    return out, gate
